# baseline (speedup 1.0000x reference)
import functools

import jax
import jax.numpy as jnp
from jax import lax
from jax.experimental import pallas as pl
from jax.experimental.pallas import tpu as pltpu
from jax.experimental.pallas import tpu_sc as plsc

_NUM_CORES = 2
_NUM_SUBCORES = 16
_NW = _NUM_CORES * _NUM_SUBCORES

_LANE = 128


def _build(batch, seq, dim):
  b_per_w = batch // _NW
  n_per_w = b_per_w * seq
  rest = seq - _LANE
  mesh = plsc.VectorSubcoreMesh(core_axis_name="c", subcore_axis_name="s")

  @functools.partial(
      pl.kernel,
      out_type=jax.ShapeDtypeStruct((batch, seq, dim), jnp.float32),
      mesh=mesh,
      scratch_types=[
          pltpu.VMEM((n_per_w,), jnp.int32),
          pltpu.VMEM((2, seq, dim), jnp.float32),
          pltpu.SemaphoreType.DMA,
          pltpu.SemaphoreType.DMA((2,)),
          pltpu.SemaphoreType.DMA((2,)),
      ],
      compiler_params=pltpu.CompilerParams(use_tc_tiling_on_sc=False),
  )
  def lookup(ids_hbm, table_hbm, out_hbm, idx_v, g_v, isem, gsem, osem):
    wid = lax.axis_index("s") * _NUM_CORES + lax.axis_index("c")
    base = wid * n_per_w
    b0 = wid * b_per_w

    pltpu.async_copy(ids_hbm.at[pl.ds(base, n_per_w)], idx_v, isem)
    pltpu.make_async_copy(ids_hbm.at[pl.ds(base, n_per_w)], idx_v,
                          isem).wait()

    def gathers(k, s):
      return (
          pltpu.make_async_copy(
              table_hbm.at[idx_v.at[pl.ds(k * seq, _LANE)]],
              g_v.at[s].at[pl.ds(0, _LANE)], gsem.at[s]),
          pltpu.make_async_copy(
              table_hbm.at[idx_v.at[pl.ds(k * seq + _LANE, rest)]],
              g_v.at[s].at[pl.ds(_LANE, rest)], gsem.at[s]),
      )

    def writeout(k, s):
      return pltpu.make_async_copy(g_v.at[s], out_hbm.at[b0 + k],
                                   osem.at[s])

    for c in gathers(0, 0):
      c.start()

    def body(t, carry):
      for j in range(2):
        k = t * 2 + j
        s = j
        for c in gathers(k, s):
          c.wait()
        writeout(k, s).start()
        @pl.when(k + 1 < b_per_w)
        def _():
          @pl.when(k >= 1)
          def _():
            writeout(k - 1, 1 - s).wait()
          for c in gathers(k + 1, 1 - s):
            c.start()
      return carry

    lax.fori_loop(0, b_per_w // 2, body, 0)

    writeout(b_per_w - 2, 0).wait()
    writeout(b_per_w - 1, 1).wait()

  return lookup


def kernel(token_ids, W):
  b, s = token_ids.shape
  _, dim = W.shape
  ids = token_ids.reshape(b * s).astype(jnp.int32)
  return _build(b, s, dim)(ids, W)
